# Initial kernel scaffold; baseline (speedup 1.0000x reference)
import jax
import jax.numpy as jnp
from jax.experimental import pallas as pl


def kernel(x, edge_index, W1, b1, g1, bt1, W2, b2, g2, bt2, W3, b3, g3, bt3, W4, b4):
    raise NotImplementedError("write your pallas kernel here")


# speedup vs baseline: 10.5788x; 10.5788x over previous
import functools

import jax
import jax.numpy as jnp
from jax import lax
from jax.experimental import pallas as pl
from jax.experimental.pallas import tpu as pltpu
from jax.experimental.pallas import tpu_sc as plsc

N = 10000
E = 320000
H = 128
CH = 80
EPC = E // 2
EPT = EPC // 16
NCH = EPT // CH
ROWS_A = 640
ROWS_B = N - 15 * ROWS_A
BNSCALE = 0.9999950000374996

_mesh = plsc.VectorSubcoreMesh(core_axis_name="c", subcore_axis_name="s")


def _sc_degree(col, zeros128, ones80):
    @functools.partial(
        pl.kernel,
        mesh=_mesh,
        out_type=jax.ShapeDtypeStruct((2, N, H), jnp.float32),
        scratch_types=[
            pltpu.VMEM((CH,), jnp.int32),
            pltpu.VMEM((CH, H), jnp.float32),
            pltpu.VMEM_SHARED((N, H), jnp.float32),
        ],
    )
    def k(c_hbm, z_hbm, ones_hbm, out_hbm, cidx, ones_v, acc):
        c = lax.axis_index("c")
        s = lax.axis_index("s")
        pltpu.sync_copy(ones_hbm, ones_v)

        @pl.when(s < 15)
        def _():
            pltpu.sync_copy(z_hbm.at[pl.ds(s * ROWS_A, ROWS_A)],
                            acc.at[pl.ds(s * ROWS_A, ROWS_A)])

        @pl.when(s == 15)
        def _():
            pltpu.sync_copy(z_hbm.at[pl.ds(15 * ROWS_A, ROWS_B)],
                            acc.at[pl.ds(15 * ROWS_A, ROWS_B)])

        plsc.subcore_barrier()
        ebase = c * EPC + s * EPT

        def body(i, carry):
            pltpu.sync_copy(c_hbm.at[pl.ds(ebase + i * CH, CH)], cidx)
            pltpu.sync_copy(ones_v, acc.at[cidx], add=True)
            return carry

        lax.fori_loop(0, NCH, body, 0)
        plsc.subcore_barrier()

        @pl.when(s < 15)
        def _():
            pltpu.sync_copy(acc.at[pl.ds(s * ROWS_A, ROWS_A)],
                            out_hbm.at[c].at[pl.ds(s * ROWS_A, ROWS_A)])

        @pl.when(s == 15)
        def _():
            pltpu.sync_copy(acc.at[pl.ds(15 * ROWS_A, ROWS_B)],
                            out_hbm.at[c].at[pl.ds(15 * ROWS_A, ROWS_B)])

    return k(col, zeros128, ones80)


def _sc_conv(u, zeros128, row, col):
    @functools.partial(
        pl.kernel,
        mesh=_mesh,
        out_type=jax.ShapeDtypeStruct((2, N, H), jnp.float32),
        scratch_types=[
            pltpu.VMEM((CH,), jnp.int32),
            pltpu.VMEM((CH,), jnp.int32),
            pltpu.VMEM((CH, H), jnp.float32),
            pltpu.VMEM_SHARED((N, H), jnp.float32),
            pltpu.SemaphoreType.DMA,
        ],
    )
    def k(u_hbm, z_hbm, r_hbm, c_hbm, out_hbm, ridx, cidx, rows, acc, sem):
        c = lax.axis_index("c")
        s = lax.axis_index("s")

        @pl.when(c == 0)
        def _():
            @pl.when(s < 15)
            def _():
                pltpu.sync_copy(u_hbm.at[pl.ds(s * ROWS_A, ROWS_A)],
                                acc.at[pl.ds(s * ROWS_A, ROWS_A)])

            @pl.when(s == 15)
            def _():
                pltpu.sync_copy(u_hbm.at[pl.ds(15 * ROWS_A, ROWS_B)],
                                acc.at[pl.ds(15 * ROWS_A, ROWS_B)])

        @pl.when(c == 1)
        def _():
            @pl.when(s < 15)
            def _():
                pltpu.sync_copy(z_hbm.at[pl.ds(s * ROWS_A, ROWS_A)],
                                acc.at[pl.ds(s * ROWS_A, ROWS_A)])

            @pl.when(s == 15)
            def _():
                pltpu.sync_copy(z_hbm.at[pl.ds(15 * ROWS_A, ROWS_B)],
                                acc.at[pl.ds(15 * ROWS_A, ROWS_B)])

        plsc.subcore_barrier()
        ebase = c * EPC + s * EPT

        def body(i, carry):
            pltpu.sync_copy(r_hbm.at[pl.ds(ebase + i * CH, CH)], ridx)
            pltpu.sync_copy(c_hbm.at[pl.ds(ebase + i * CH, CH)], cidx)
            pltpu.async_copy(u_hbm.at[ridx], rows, sem).wait()
            pltpu.sync_copy(rows, acc.at[cidx], add=True)
            return carry

        lax.fori_loop(0, NCH, body, 0)
        plsc.subcore_barrier()

        @pl.when(s < 15)
        def _():
            pltpu.sync_copy(acc.at[pl.ds(s * ROWS_A, ROWS_A)],
                            out_hbm.at[c].at[pl.ds(s * ROWS_A, ROWS_A)])

        @pl.when(s == 15)
        def _():
            pltpu.sync_copy(acc.at[pl.ds(15 * ROWS_A, ROWS_B)],
                            out_hbm.at[c].at[pl.ds(15 * ROWS_A, ROWS_B)])

    return k(u, zeros128, row, col)


def _dot(a, b):
    return jnp.dot(a, b, preferred_element_type=jnp.float32)


def _dinv(d_ref):
    return lax.rsqrt(1.0 + d_ref[0] + d_ref[1])


def _tc_first(x, deg2, W1):
    def body(x_ref, d_ref, w_ref, u_ref):
        u_ref[...] = _dinv(d_ref) * _dot(x_ref[...], w_ref[...])

    return pl.pallas_call(
        body, out_shape=jax.ShapeDtypeStruct((N, H), jnp.float32),
    )(x, deg2, W1)


def _tc_mid(S, deg2, W, b_prev, g_prev, bt_prev):
    def body(S_ref, d_ref, w_ref, b_ref, g_ref, bt_ref, x_ref, u_ref):
        dinv = _dinv(d_ref)
        agg = dinv * (S_ref[0] + S_ref[1]) + b_ref[...]
        h = jnp.maximum(agg * (g_ref[...] * BNSCALE) + bt_ref[...], 0.0)
        x_ref[...] = h
        u_ref[...] = dinv * _dot(h, w_ref[...])

    return pl.pallas_call(
        body,
        out_shape=(jax.ShapeDtypeStruct((N, H), jnp.float32),
                   jax.ShapeDtypeStruct((N, H), jnp.float32)),
    )(S, deg2, W, b_prev, g_prev, bt_prev)


def _tc_last(S3, x1, deg2, b3, g3, bt3):
    def body(S_ref, x1_ref, d_ref, b_ref, g_ref, bt_ref, v_ref):
        dinv = _dinv(d_ref)
        agg = dinv * (S_ref[0] + S_ref[1]) + b_ref[...]
        h = jnp.maximum(agg * (g_ref[...] * BNSCALE) + bt_ref[...], 0.0)
        v_ref[...] = dinv * (h + x1_ref[...])

    return pl.pallas_call(
        body, out_shape=jax.ShapeDtypeStruct((N, H), jnp.float32),
    )(S3, x1, deg2, b3, g3, bt3)


def _tc_out(T, deg2, W4t, b4):
    def body(T_ref, d_ref, w_ref, b_ref, o_ref):
        Tf = T_ref[0] + T_ref[1]
        o_ref[...] = (_dinv(d_ref)
                      * jnp.sum(Tf * w_ref[...], axis=1, keepdims=True)
                      + b_ref[...])

    return pl.pallas_call(
        body, out_shape=jax.ShapeDtypeStruct((N, 1), jnp.float32),
    )(T, deg2, W4t, b4)


def kernel(x, edge_index, W1, b1, g1, bt1, W2, b2, g2, bt2, W3, b3, g3, bt3,
           W4, b4):
    row = edge_index[0]
    col = edge_index[1]
    zeros128 = jnp.zeros((N, H), jnp.float32)
    ones80 = jnp.ones((CH, H), jnp.float32)

    P = _sc_degree(col, zeros128, ones80)
    deg2 = P[:, :, 0:1]

    u1 = _tc_first(x, deg2, W1)
    S1 = _sc_conv(u1, zeros128, row, col)
    x1, u2 = _tc_mid(S1, deg2, W2, b1.reshape(1, H), g1.reshape(1, H),
                     bt1.reshape(1, H))
    S2 = _sc_conv(u2, zeros128, row, col)
    _, u3 = _tc_mid(S2, deg2, W3, b2.reshape(1, H), g2.reshape(1, H),
                    bt2.reshape(1, H))
    S3 = _sc_conv(u3, zeros128, row, col)
    v = _tc_last(S3, x1, deg2, b3.reshape(1, H), g3.reshape(1, H),
                 bt3.reshape(1, H))
    T = _sc_conv(v, zeros128, row, col)
    return _tc_out(T, deg2, W4.reshape(1, H), b4.reshape(1, 1))


# speedup vs baseline: 20.5407x; 1.9417x over previous
import functools

import jax
import jax.numpy as jnp
from jax import lax
from jax.experimental import pallas as pl
from jax.experimental.pallas import tpu as pltpu
from jax.experimental.pallas import tpu_sc as plsc

N = 10000
E = 320000
H = 128
CH = 40
EPC = E // 2
EPT = EPC // 16
NCH = EPT // CH
ROWS_A = 640
ROWS_B = N - 15 * ROWS_A
BNSCALE = 0.9999950000374996

_mesh = plsc.VectorSubcoreMesh(core_axis_name="c", subcore_axis_name="s")


NB = 5
NG = NCH // NB


def _sc_degree(eint, zeros128, ones80):
    @functools.partial(
        pl.kernel,
        mesh=_mesh,
        out_type=jax.ShapeDtypeStruct((2, N, H), jnp.float32),
        scratch_types=[
            pltpu.VMEM((NB, 2, CH), jnp.int32),
            pltpu.VMEM((CH, H), jnp.float32),
            pltpu.VMEM_SHARED((N, H), jnp.float32),
            pltpu.SemaphoreType.DMA((NB,)),
            pltpu.SemaphoreType.DMA((NB,)),
        ],
    )
    def k(e_hbm, z_hbm, ones_hbm, out_hbm, idxr, ones_v, acc, isem, ssem):
        c = lax.axis_index("c")
        s = lax.axis_index("s")
        pltpu.sync_copy(ones_hbm, ones_v)
        cb = c * (E // 2 // CH) + s * NCH

        @pl.when(s < 15)
        def _():
            pltpu.sync_copy(z_hbm.at[pl.ds(s * ROWS_A, ROWS_A)],
                            acc.at[pl.ds(s * ROWS_A, ROWS_A)])

        @pl.when(s == 15)
        def _():
            pltpu.sync_copy(z_hbm.at[pl.ds(15 * ROWS_A, ROWS_B)],
                            acc.at[pl.ds(15 * ROWS_A, ROWS_B)])

        plsc.subcore_barrier()

        for j in range(NB):
            pltpu.async_copy(e_hbm.at[cb + j], idxr.at[j], isem.at[j])

        def body(p, carry):
            for j in range(NB):
                pltpu.make_async_copy(e_hbm.at[cb], idxr.at[j],
                                      isem.at[j]).wait()
                pltpu.async_copy(ones_v, acc.at[idxr.at[j, 1]],
                                 ssem.at[j], add=True)
            for j in range(NB):
                pltpu.make_async_copy(ones_v, acc.at[idxr.at[j, 1]],
                                      ssem.at[j]).wait()

                @pl.when(p < NG - 1)
                def _():
                    nck = (p + 1) * NB + j
                    pltpu.async_copy(e_hbm.at[cb + nck], idxr.at[j],
                                     isem.at[j])
            return carry

        lax.fori_loop(0, NG, body, 0)
        plsc.subcore_barrier()

        @pl.when(s < 15)
        def _():
            pltpu.sync_copy(acc.at[pl.ds(s * ROWS_A, ROWS_A)],
                            out_hbm.at[c].at[pl.ds(s * ROWS_A, ROWS_A)])

        @pl.when(s == 15)
        def _():
            pltpu.sync_copy(acc.at[pl.ds(15 * ROWS_A, ROWS_B)],
                            out_hbm.at[c].at[pl.ds(15 * ROWS_A, ROWS_B)])

    return k(eint, zeros128, ones80)


def _sc_conv(u, zeros128, eint):
    @functools.partial(
        pl.kernel,
        mesh=_mesh,
        out_type=jax.ShapeDtypeStruct((2, N, H), jnp.float32),
        scratch_types=[
            pltpu.VMEM((NB, 2, CH), jnp.int32),
            pltpu.VMEM((NB, CH, H), jnp.float32),
            pltpu.VMEM_SHARED((N, H), jnp.float32),
            pltpu.SemaphoreType.DMA((NB,)),
            pltpu.SemaphoreType.DMA((NB,)),
            pltpu.SemaphoreType.DMA((NB,)),
        ],
    )
    def k(u_hbm, z_hbm, e_hbm, out_hbm, idxr, rows, acc, isem, gsem, ssem):
        c = lax.axis_index("c")
        s = lax.axis_index("s")
        cb = c * (E // 2 // CH) + s * NCH

        @pl.when(c == 0)
        def _():
            @pl.when(s < 15)
            def _():
                pltpu.sync_copy(u_hbm.at[pl.ds(s * ROWS_A, ROWS_A)],
                                acc.at[pl.ds(s * ROWS_A, ROWS_A)])

            @pl.when(s == 15)
            def _():
                pltpu.sync_copy(u_hbm.at[pl.ds(15 * ROWS_A, ROWS_B)],
                                acc.at[pl.ds(15 * ROWS_A, ROWS_B)])

        @pl.when(c == 1)
        def _():
            @pl.when(s < 15)
            def _():
                pltpu.sync_copy(z_hbm.at[pl.ds(s * ROWS_A, ROWS_A)],
                                acc.at[pl.ds(s * ROWS_A, ROWS_A)])

            @pl.when(s == 15)
            def _():
                pltpu.sync_copy(z_hbm.at[pl.ds(15 * ROWS_A, ROWS_B)],
                                acc.at[pl.ds(15 * ROWS_A, ROWS_B)])

        plsc.subcore_barrier()

        for j in range(NB):
            pltpu.async_copy(e_hbm.at[cb + j], idxr.at[j], isem.at[j])

        def body(p, carry):
            for j in range(NB):
                pltpu.make_async_copy(e_hbm.at[cb], idxr.at[j],
                                      isem.at[j]).wait()
                pltpu.async_copy(u_hbm.at[idxr.at[j, 0]], rows.at[j],
                                 gsem.at[j])
            for j in range(NB):
                pltpu.make_async_copy(u_hbm.at[idxr.at[j, 0]], rows.at[j],
                                      gsem.at[j]).wait()
                pltpu.async_copy(rows.at[j], acc.at[idxr.at[j, 1]],
                                 ssem.at[j], add=True)
            for j in range(NB):
                pltpu.make_async_copy(rows.at[j], acc.at[idxr.at[j, 1]],
                                      ssem.at[j]).wait()

                @pl.when(p < NG - 1)
                def _():
                    nck = (p + 1) * NB + j
                    pltpu.async_copy(e_hbm.at[cb + nck], idxr.at[j],
                                     isem.at[j])

            return carry

        lax.fori_loop(0, NG, body, 0)
        plsc.subcore_barrier()

        @pl.when(s < 15)
        def _():
            pltpu.sync_copy(acc.at[pl.ds(s * ROWS_A, ROWS_A)],
                            out_hbm.at[c].at[pl.ds(s * ROWS_A, ROWS_A)])

        @pl.when(s == 15)
        def _():
            pltpu.sync_copy(acc.at[pl.ds(15 * ROWS_A, ROWS_B)],
                            out_hbm.at[c].at[pl.ds(15 * ROWS_A, ROWS_B)])

    return k(u, zeros128, eint)


def _dot(a, b):
    return jnp.dot(a, b, preferred_element_type=jnp.float32)


def _dinv(d_ref):
    return lax.rsqrt(1.0 + d_ref[0] + d_ref[1])


def _tc_first(x, deg2, W1):
    def body(x_ref, d_ref, w_ref, u_ref):
        u_ref[...] = _dinv(d_ref) * _dot(x_ref[...], w_ref[...])

    return pl.pallas_call(
        body, out_shape=jax.ShapeDtypeStruct((N, H), jnp.float32),
    )(x, deg2, W1)


def _tc_mid(S, deg2, W, b_prev, g_prev, bt_prev):
    def body(S_ref, d_ref, w_ref, b_ref, g_ref, bt_ref, x_ref, u_ref):
        dinv = _dinv(d_ref)
        agg = dinv * (S_ref[0] + S_ref[1]) + b_ref[...]
        h = jnp.maximum(agg * (g_ref[...] * BNSCALE) + bt_ref[...], 0.0)
        x_ref[...] = h
        u_ref[...] = dinv * _dot(h, w_ref[...])

    return pl.pallas_call(
        body,
        out_shape=(jax.ShapeDtypeStruct((N, H), jnp.float32),
                   jax.ShapeDtypeStruct((N, H), jnp.float32)),
    )(S, deg2, W, b_prev, g_prev, bt_prev)


def _tc_last(S3, x1, deg2, b3, g3, bt3):
    def body(S_ref, x1_ref, d_ref, b_ref, g_ref, bt_ref, v_ref):
        dinv = _dinv(d_ref)
        agg = dinv * (S_ref[0] + S_ref[1]) + b_ref[...]
        h = jnp.maximum(agg * (g_ref[...] * BNSCALE) + bt_ref[...], 0.0)
        v_ref[...] = dinv * (h + x1_ref[...])

    return pl.pallas_call(
        body, out_shape=jax.ShapeDtypeStruct((N, H), jnp.float32),
    )(S3, x1, deg2, b3, g3, bt3)


def _tc_out(T, deg2, W4t, b4):
    def body(T_ref, d_ref, w_ref, b_ref, o_ref):
        Tf = T_ref[0] + T_ref[1]
        o_ref[...] = (_dinv(d_ref)
                      * jnp.sum(Tf * w_ref[...], axis=1, keepdims=True)
                      + b_ref[...])

    return pl.pallas_call(
        body, out_shape=jax.ShapeDtypeStruct((N, 1), jnp.float32),
    )(T, deg2, W4t, b4)


def kernel(x, edge_index, W1, b1, g1, bt1, W2, b2, g2, bt2, W3, b3, g3, bt3,
           W4, b4):
    row = edge_index[0]
    col = edge_index[1]
    eint = jnp.stack([row.reshape(E // CH, CH), col.reshape(E // CH, CH)],
                     axis=1)
    zeros128 = jnp.zeros((N, H), jnp.float32)
    ones80 = jnp.ones((CH, H), jnp.float32)

    P = _sc_degree(eint, zeros128, ones80)
    deg2 = P[:, :, 0:1]

    u1 = _tc_first(x, deg2, W1)
    S1 = _sc_conv(u1, zeros128, eint)
    x1, u2 = _tc_mid(S1, deg2, W2, b1.reshape(1, H), g1.reshape(1, H),
                     bt1.reshape(1, H))
    S2 = _sc_conv(u2, zeros128, eint)
    _, u3 = _tc_mid(S2, deg2, W3, b2.reshape(1, H), g2.reshape(1, H),
                    bt2.reshape(1, H))
    S3 = _sc_conv(u3, zeros128, eint)
    v = _tc_last(S3, x1, deg2, b3.reshape(1, H), g3.reshape(1, H),
                 bt3.reshape(1, H))
    T = _sc_conv(v, zeros128, eint)
    return _tc_out(T, deg2, W4.reshape(1, H), b4.reshape(1, 1))


# speedup vs baseline: 21.0979x; 1.0271x over previous
import functools

import jax
import jax.numpy as jnp
from jax import lax
from jax.experimental import pallas as pl
from jax.experimental.pallas import tpu as pltpu
from jax.experimental.pallas import tpu_sc as plsc

N = 10000
E = 320000
H = 128
CH = 40
EPC = E // 2
EPT = EPC // 16
NCH = EPT // CH
ROWS_A = 640
ROWS_B = N - 15 * ROWS_A
BNSCALE = 0.9999950000374996

_mesh = plsc.VectorSubcoreMesh(core_axis_name="c", subcore_axis_name="s")


NB = 5
NG = NCH // NB


def _sc_degree(eint, zeros128, ones80):
    @functools.partial(
        pl.kernel,
        mesh=_mesh,
        out_type=jax.ShapeDtypeStruct((2, N, 16), jnp.float32),
        scratch_types=[
            pltpu.VMEM((NB, 2, CH), jnp.int32),
            pltpu.VMEM((CH, 16), jnp.float32),
            pltpu.VMEM_SHARED((N, 16), jnp.float32),
            pltpu.SemaphoreType.DMA((NB,)),
            pltpu.SemaphoreType.DMA((NB,)),
        ],
    )
    def k(e_hbm, z_hbm, ones_hbm, out_hbm, idxr, ones_v, acc, isem, ssem):
        c = lax.axis_index("c")
        s = lax.axis_index("s")
        pltpu.sync_copy(ones_hbm, ones_v)
        cb = c * (E // 2 // CH) + s * NCH

        @pl.when(s < 15)
        def _():
            pltpu.sync_copy(z_hbm.at[pl.ds(s * ROWS_A, ROWS_A)],
                            acc.at[pl.ds(s * ROWS_A, ROWS_A)])

        @pl.when(s == 15)
        def _():
            pltpu.sync_copy(z_hbm.at[pl.ds(15 * ROWS_A, ROWS_B)],
                            acc.at[pl.ds(15 * ROWS_A, ROWS_B)])

        plsc.subcore_barrier()

        for j in range(NB):
            pltpu.async_copy(e_hbm.at[cb + j], idxr.at[j], isem.at[j])

        def body(p, carry):
            for j in range(NB):
                pltpu.make_async_copy(e_hbm.at[cb], idxr.at[j],
                                      isem.at[j]).wait()
                pltpu.async_copy(ones_v, acc.at[idxr.at[j, 1]],
                                 ssem.at[j], add=True)
            for j in range(NB):
                pltpu.make_async_copy(ones_v, acc.at[idxr.at[j, 1]],
                                      ssem.at[j]).wait()

                @pl.when(p < NG - 1)
                def _():
                    nck = (p + 1) * NB + j
                    pltpu.async_copy(e_hbm.at[cb + nck], idxr.at[j],
                                     isem.at[j])
            return carry

        lax.fori_loop(0, NG, body, 0)
        plsc.subcore_barrier()

        @pl.when(s < 15)
        def _():
            pltpu.sync_copy(acc.at[pl.ds(s * ROWS_A, ROWS_A)],
                            out_hbm.at[c].at[pl.ds(s * ROWS_A, ROWS_A)])

        @pl.when(s == 15)
        def _():
            pltpu.sync_copy(acc.at[pl.ds(15 * ROWS_A, ROWS_B)],
                            out_hbm.at[c].at[pl.ds(15 * ROWS_A, ROWS_B)])

    return k(eint, zeros128, ones80)


def _sc_conv(u, zeros128, eint):
    @functools.partial(
        pl.kernel,
        mesh=_mesh,
        out_type=jax.ShapeDtypeStruct((2, N, H), jnp.float32),
        scratch_types=[
            pltpu.VMEM((NB, 2, CH), jnp.int32),
            pltpu.VMEM((NB, CH, H), jnp.float32),
            pltpu.VMEM_SHARED((N, H), jnp.float32),
            pltpu.SemaphoreType.DMA((NB,)),
            pltpu.SemaphoreType.DMA((NB,)),
            pltpu.SemaphoreType.DMA((NB,)),
        ],
    )
    def k(u_hbm, ui_hbm, z_hbm, e_hbm, out_hbm, idxr, rows, acc, isem, gsem, ssem):
        c = lax.axis_index("c")
        s = lax.axis_index("s")
        cb = c * (E // 2 // CH) + s * NCH

        @pl.when(c == 0)
        def _():
            @pl.when(s < 15)
            def _():
                pltpu.sync_copy(ui_hbm.at[pl.ds(s * ROWS_A, ROWS_A)],
                                acc.at[pl.ds(s * ROWS_A, ROWS_A)])

            @pl.when(s == 15)
            def _():
                pltpu.sync_copy(ui_hbm.at[pl.ds(15 * ROWS_A, ROWS_B)],
                                acc.at[pl.ds(15 * ROWS_A, ROWS_B)])

        @pl.when(c == 1)
        def _():
            @pl.when(s < 15)
            def _():
                pltpu.sync_copy(z_hbm.at[pl.ds(s * ROWS_A, ROWS_A)],
                                acc.at[pl.ds(s * ROWS_A, ROWS_A)])

            @pl.when(s == 15)
            def _():
                pltpu.sync_copy(z_hbm.at[pl.ds(15 * ROWS_A, ROWS_B)],
                                acc.at[pl.ds(15 * ROWS_A, ROWS_B)])

        plsc.subcore_barrier()

        for j in range(NB):
            pltpu.async_copy(e_hbm.at[cb + j], idxr.at[j], isem.at[j])

        def body(p, carry):
            for j in range(NB):
                pltpu.make_async_copy(e_hbm.at[cb], idxr.at[j],
                                      isem.at[j]).wait()
                pltpu.async_copy(u_hbm.at[idxr.at[j, 0]], rows.at[j],
                                 gsem.at[j])
            for j in range(NB):
                pltpu.make_async_copy(u_hbm.at[idxr.at[j, 0]], rows.at[j],
                                      gsem.at[j]).wait()
                pltpu.async_copy(rows.at[j], acc.at[idxr.at[j, 1]],
                                 ssem.at[j], add=True)
            for j in range(NB):
                pltpu.make_async_copy(rows.at[j], acc.at[idxr.at[j, 1]],
                                      ssem.at[j]).wait()

                @pl.when(p < NG - 1)
                def _():
                    nck = (p + 1) * NB + j
                    pltpu.async_copy(e_hbm.at[cb + nck], idxr.at[j],
                                     isem.at[j])

            return carry

        lax.fori_loop(0, NG, body, 0)
        plsc.subcore_barrier()

        @pl.when(s < 15)
        def _():
            pltpu.sync_copy(acc.at[pl.ds(s * ROWS_A, ROWS_A)],
                            out_hbm.at[c].at[pl.ds(s * ROWS_A, ROWS_A)])

        @pl.when(s == 15)
        def _():
            pltpu.sync_copy(acc.at[pl.ds(15 * ROWS_A, ROWS_B)],
                            out_hbm.at[c].at[pl.ds(15 * ROWS_A, ROWS_B)])

    return k(u, u, zeros128, eint)


def _dot(a, b):
    return jnp.dot(a, b, preferred_element_type=jnp.float32)


def _dinv(d_ref):
    return lax.rsqrt(1.0 + d_ref[0] + d_ref[1])


def _tc_first(x, deg2, W1):
    def body(x_ref, d_ref, w_ref, u_ref):
        u_ref[...] = _dinv(d_ref) * _dot(x_ref[...], w_ref[...])

    return pl.pallas_call(
        body, out_shape=jax.ShapeDtypeStruct((N, H), jnp.float32),
    )(x, deg2, W1)


def _tc_mid(S, deg2, W, b_prev, g_prev, bt_prev):
    def body(S_ref, d_ref, w_ref, b_ref, g_ref, bt_ref, x_ref, u_ref):
        dinv = _dinv(d_ref)
        agg = dinv * (S_ref[0] + S_ref[1]) + b_ref[...]
        h = jnp.maximum(agg * (g_ref[...] * BNSCALE) + bt_ref[...], 0.0)
        x_ref[...] = h
        u_ref[...] = dinv * _dot(h, w_ref[...])

    return pl.pallas_call(
        body,
        out_shape=(jax.ShapeDtypeStruct((N, H), jnp.float32),
                   jax.ShapeDtypeStruct((N, H), jnp.float32)),
    )(S, deg2, W, b_prev, g_prev, bt_prev)


def _tc_last(S3, x1, deg2, b3, g3, bt3):
    def body(S_ref, x1_ref, d_ref, b_ref, g_ref, bt_ref, v_ref):
        dinv = _dinv(d_ref)
        agg = dinv * (S_ref[0] + S_ref[1]) + b_ref[...]
        h = jnp.maximum(agg * (g_ref[...] * BNSCALE) + bt_ref[...], 0.0)
        v_ref[...] = dinv * (h + x1_ref[...])

    return pl.pallas_call(
        body, out_shape=jax.ShapeDtypeStruct((N, H), jnp.float32),
    )(S3, x1, deg2, b3, g3, bt3)


def _tc_out(T, deg2, W4t, b4):
    def body(T_ref, d_ref, w_ref, b_ref, o_ref):
        Tf = T_ref[0] + T_ref[1]
        o_ref[...] = (_dinv(d_ref)
                      * jnp.sum(Tf * w_ref[...], axis=1, keepdims=True)
                      + b_ref[...])

    return pl.pallas_call(
        body, out_shape=jax.ShapeDtypeStruct((N, 1), jnp.float32),
    )(T, deg2, W4t, b4)


def kernel(x, edge_index, W1, b1, g1, bt1, W2, b2, g2, bt2, W3, b3, g3, bt3,
           W4, b4):
    row = edge_index[0]
    col = edge_index[1]
    eint = jnp.stack([row.reshape(E // CH, CH), col.reshape(E // CH, CH)],
                     axis=1)
    zeros128 = jnp.zeros((N, H), jnp.float32)
    ones80 = jnp.ones((CH, 16), jnp.float32)

    P = _sc_degree(eint, jnp.zeros((N, 16), jnp.float32), ones80)
    deg2 = P[:, :, 0:1]

    u1 = _tc_first(x, deg2, W1)
    S1 = _sc_conv(u1, zeros128, eint)
    x1, u2 = _tc_mid(S1, deg2, W2, b1.reshape(1, H), g1.reshape(1, H),
                     bt1.reshape(1, H))
    S2 = _sc_conv(u2, zeros128, eint)
    _, u3 = _tc_mid(S2, deg2, W3, b2.reshape(1, H), g2.reshape(1, H),
                    bt2.reshape(1, H))
    S3 = _sc_conv(u3, zeros128, eint)
    v = _tc_last(S3, x1, deg2, b3.reshape(1, H), g3.reshape(1, H),
                 bt3.reshape(1, H))
    T = _sc_conv(v, zeros128, eint)
    return _tc_out(T, deg2, W4.reshape(1, H), b4.reshape(1, 1))
